# baseline (speedup 1.0000x reference)
import functools

import jax
import jax.numpy as jnp
from jax import lax
from jax.experimental import pallas as pl
from jax.experimental.pallas import tpu as pltpu
from jax.experimental.pallas import tpu_sc as plsc

NC = 2
NS = 16
LANES = 16
CHUNK = 64
NBUF = 4


def _make_sc_kernel(n_nodes, n_words, e_pad, n_edges):
    e_tile = e_pad // NS
    n_chunks = e_tile // CHUNK
    assert n_chunks % NBUF == 0
    tail = n_edges - (NS - 1) * e_tile
    assert 0 < tail <= e_tile and tail % 8 == 0

    mesh = plsc.VectorSubcoreMesh(core_axis_name="c", subcore_axis_name="s")
    stage_rows = (n_nodes // (8 * NS)) * 8
    stage_rem = n_nodes - stage_rows * NS

    @functools.partial(
        pl.kernel,
        out_type=jax.ShapeDtypeStruct((NC, n_edges), jnp.float32),
        mesh=mesh,
        compiler_params=pltpu.CompilerParams(
            use_tc_tiling_on_sc=False, needs_layout_passes=False),
        scratch_types=[
            pltpu.VMEM((e_tile,), jnp.int32),
            pltpu.VMEM((e_tile,), jnp.int32),
            pltpu.VMEM((e_tile,), jnp.float32),
            [pltpu.VMEM((CHUNK, n_words), jnp.float32) for _ in range(NBUF)],
            [pltpu.VMEM((CHUNK, n_words), jnp.float32) for _ in range(NBUF)],
            [pltpu.SemaphoreType.DMA for _ in range(NBUF)],
            pltpu.VMEM_SHARED((n_nodes, n_words), jnp.float32),
        ],
    )
    def sc_kernel(x_hbm, src_hbm, dst_hbm, out_hbm,
                  src_v, dst_v, out_v, bus, bvs, sems, xs):
        sid = lax.axis_index("s")
        cid = lax.axis_index("c")
        base = pl.multiple_of(sid * e_tile, 8)

        r0 = pl.multiple_of(sid * stage_rows, 8)
        c0 = pl.multiple_of(cid * n_words, 8)
        pltpu.sync_copy(x_hbm.at[pl.ds(r0, stage_rows), pl.ds(c0, n_words)],
                        xs.at[pl.ds(r0, stage_rows)])
        if stage_rem:
            t0 = stage_rows * NS

            @pl.when(sid == 0)
            def _():
                pltpu.sync_copy(
                    x_hbm.at[pl.ds(t0, stage_rem), pl.ds(c0, n_words)],
                    xs.at[pl.ds(t0, stage_rem)])

        if tail == e_tile:
            pltpu.sync_copy(src_hbm.at[pl.ds(base, e_tile)], src_v)
            pltpu.sync_copy(dst_hbm.at[pl.ds(base, e_tile)], dst_v)
        else:
            @pl.when(sid < NS - 1)
            def _():
                pltpu.sync_copy(src_hbm.at[pl.ds(base, e_tile)], src_v)
                pltpu.sync_copy(dst_hbm.at[pl.ds(base, e_tile)], dst_v)

            @pl.when(sid == NS - 1)
            def _():
                pltpu.sync_copy(src_hbm.at[pl.ds(base, tail)],
                                src_v.at[pl.ds(0, tail)])
                pltpu.sync_copy(dst_hbm.at[pl.ds(base, tail)],
                                dst_v.at[pl.ds(0, tail)])
                zeros16 = jnp.zeros((LANES,), jnp.int32)

                def zfill(i, carry):
                    off = pl.multiple_of(tail + i * LANES, 8)
                    src_v[pl.ds(off, LANES)] = zeros16
                    dst_v[pl.ds(off, LANES)] = zeros16
                    return carry

                lax.fori_loop(0, (e_tile - tail) // LANES, zfill, 0,
                              unroll=False)

        plsc.subcore_barrier()

        def fire(c, s):
            cb = pl.multiple_of(c * CHUNK, 8)
            pltpu.async_copy(xs.at[src_v.at[pl.ds(cb, CHUNK)]], bus[s], sems[s])
            pltpu.async_copy(xs.at[dst_v.at[pl.ds(cb, CHUNK)]], bvs[s], sems[s])

        def drain(s):
            pltpu.make_async_copy(
                xs.at[src_v.at[pl.ds(0, CHUNK)]], bus[s], sems[s]).wait()
            pltpu.make_async_copy(
                xs.at[dst_v.at[pl.ds(0, CHUNK)]], bvs[s], sems[s]).wait()

        lane = lax.iota(jnp.int32, LANES)
        nk = n_words // LANES

        def compute(c, s):
            cb = c * CHUNK
            bu, bv = bus[s], bvs[s]

            def grp_body(g, carry2):
                gb = g * LANES
                vec = jnp.zeros((LANES,), jnp.float32)
                for j in range(LANES):
                    e = gb + j
                    acc = jnp.zeros((LANES,), jnp.float32)
                    for k in range(nk):
                        au = plsc.bitcast(bu[e, pl.ds(k * 16, 16)], jnp.bfloat16)
                        av = plsc.bitcast(bv[e, pl.ds(k * 16, 16)], jnp.bfloat16)
                        u0, u1 = plsc.unpack(au, format=plsc.PackFormat.INTERLEAVED)
                        v0, v1 = plsc.unpack(av, format=plsc.PackFormat.INTERLEAVED)
                        acc = acc + u0 * v0
                        acc = acc + u1 * v1
                    vec = jnp.where(lane == j, jnp.sum(acc), vec)
                out_v[pl.ds(pl.multiple_of(cb + gb, 8), LANES)] = vec
                return carry2

            lax.fori_loop(0, CHUNK // LANES, grp_body, 0, unroll=False)

        for s in range(NBUF):
            fire(s, s)

        def ring_body(q, carry):
            c0 = q * NBUF
            for s in range(NBUF):
                drain(s)
                compute(c0 + s, s)

                @pl.when(c0 + s + NBUF < n_chunks)
                def _():
                    fire(c0 + s + NBUF, s)

            return carry

        lax.fori_loop(0, n_chunks // NBUF, ring_body, 0, unroll=False)
        if tail == e_tile:
            pltpu.sync_copy(out_v, out_hbm.at[cid, pl.ds(base, e_tile)])
        else:
            @pl.when(sid < NS - 1)
            def _():
                pltpu.sync_copy(out_v, out_hbm.at[cid, pl.ds(base, e_tile)])

            @pl.when(sid == NS - 1)
            def _():
                pltpu.sync_copy(out_v.at[pl.ds(0, tail)],
                                out_hbm.at[cid, pl.ds(base, tail)])

    return sc_kernel


def _combine_partials(partials, n_edges):
    cols = 128
    rows = n_edges // cols

    def body(p_ref, o_ref):
        o_ref[...] = p_ref[0] + p_ref[1]

    out = pl.pallas_call(
        body,
        out_shape=jax.ShapeDtypeStruct((rows, cols), jnp.float32),
    )(partials.reshape(NC, rows, cols))
    return out.reshape(n_edges, 1)


def kernel(x, edge_index):
    n_nodes, d_model = x.shape
    n_edges = edge_index.shape[1]
    grain = NS * CHUNK * NBUF
    e_pad = ((n_edges + grain - 1) // grain) * grain
    n_words = d_model // (2 * NC)

    x_bf = jax.lax.bitcast_convert_type(
        x.astype(jnp.bfloat16).reshape(n_nodes, NC * n_words, 2), jnp.float32)

    src = edge_index[0].astype(jnp.int32)
    dst = edge_index[1].astype(jnp.int32)

    partials = _make_sc_kernel(n_nodes, n_words, e_pad, n_edges)(x_bf, src, dst)
    return _combine_partials(partials, n_edges)


# speedup vs baseline: 1.4457x; 1.4457x over previous
import functools

import jax
import jax.numpy as jnp
from jax import lax
from jax.experimental import pallas as pl
from jax.experimental.pallas import tpu as pltpu
from jax.experimental.pallas import tpu_sc as plsc

NC = 2
NS = 16
LANES = 16
CHUNK = 64
NBUF = 4


def _make_sc_kernel(n_nodes, n_feat, e_pad, n_edges):
    e_tile = e_pad // NS
    n_chunks = e_tile // CHUNK
    assert n_chunks % NBUF == 0
    tail = n_edges - (NS - 1) * e_tile
    assert 0 < tail <= e_tile and tail % 8 == 0

    mesh = plsc.VectorSubcoreMesh(core_axis_name="c", subcore_axis_name="s")
    stage_rows = (n_nodes // (8 * NS)) * 8
    stage_rem = n_nodes - stage_rows * NS

    @functools.partial(
        pl.kernel,
        out_type=jax.ShapeDtypeStruct((NC, n_edges), jnp.float32),
        mesh=mesh,
        compiler_params=pltpu.CompilerParams(
            use_tc_tiling_on_sc=False, needs_layout_passes=False),
        scratch_types=[
            pltpu.VMEM((e_tile,), jnp.int32),
            pltpu.VMEM((e_tile,), jnp.int32),
            pltpu.VMEM((e_tile,), jnp.float32),
            [pltpu.VMEM((CHUNK, n_feat), jnp.bfloat16) for _ in range(NBUF)],
            [pltpu.VMEM((CHUNK, n_feat), jnp.bfloat16) for _ in range(NBUF)],
            [pltpu.SemaphoreType.DMA for _ in range(NBUF)],
            pltpu.VMEM_SHARED((n_nodes, n_feat), jnp.bfloat16),
        ],
    )
    def sc_kernel(x_hbm, ei_hbm, out_hbm,
                  src_v, dst_v, out_v, bus, bvs, sems, xs):
        sid = lax.axis_index("s")
        cid = lax.axis_index("c")
        base = pl.multiple_of(sid * e_tile, 8)

        r0 = pl.multiple_of(sid * stage_rows, 8)
        c0 = pl.multiple_of(cid * n_feat, 8)
        pltpu.sync_copy(x_hbm.at[pl.ds(r0, stage_rows), pl.ds(c0, n_feat)],
                        xs.at[pl.ds(r0, stage_rows)])
        if stage_rem:
            t0 = stage_rows * NS

            @pl.when(sid == 0)
            def _():
                pltpu.sync_copy(
                    x_hbm.at[pl.ds(t0, stage_rem), pl.ds(c0, n_feat)],
                    xs.at[pl.ds(t0, stage_rem)])

        if tail == e_tile:
            pltpu.sync_copy(ei_hbm.at[0, pl.ds(base, e_tile)], src_v)
            pltpu.sync_copy(ei_hbm.at[1, pl.ds(base, e_tile)], dst_v)
        else:
            @pl.when(sid < NS - 1)
            def _():
                pltpu.sync_copy(ei_hbm.at[0, pl.ds(base, e_tile)], src_v)
                pltpu.sync_copy(ei_hbm.at[1, pl.ds(base, e_tile)], dst_v)

            @pl.when(sid == NS - 1)
            def _():
                pltpu.sync_copy(ei_hbm.at[0, pl.ds(base, tail)],
                                src_v.at[pl.ds(0, tail)])
                pltpu.sync_copy(ei_hbm.at[1, pl.ds(base, tail)],
                                dst_v.at[pl.ds(0, tail)])
                zeros16 = jnp.zeros((LANES,), jnp.int32)

                def zfill(i, carry):
                    off = pl.multiple_of(tail + i * LANES, 8)
                    src_v[pl.ds(off, LANES)] = zeros16
                    dst_v[pl.ds(off, LANES)] = zeros16
                    return carry

                lax.fori_loop(0, (e_tile - tail) // LANES, zfill, 0,
                              unroll=False)

        plsc.subcore_barrier()

        def fire(c, s):
            cb = pl.multiple_of(c * CHUNK, 8)
            pltpu.async_copy(xs.at[src_v.at[pl.ds(cb, CHUNK)]], bus[s], sems[s])
            pltpu.async_copy(xs.at[dst_v.at[pl.ds(cb, CHUNK)]], bvs[s], sems[s])

        def drain(s):
            pltpu.make_async_copy(
                xs.at[src_v.at[pl.ds(0, CHUNK)]], bus[s], sems[s]).wait()
            pltpu.make_async_copy(
                xs.at[dst_v.at[pl.ds(0, CHUNK)]], bvs[s], sems[s]).wait()

        lane = lax.iota(jnp.int32, LANES)
        nk = n_feat // 32

        def compute(c, s):
            cb = c * CHUNK
            bu, bv = bus[s], bvs[s]

            def grp_body(g, carry2):
                gb = g * LANES
                vec = jnp.zeros((LANES,), jnp.float32)
                for j in range(LANES):
                    e = gb + j
                    acc = jnp.zeros((LANES,), jnp.float32)
                    for k in range(nk):
                        au = bu[e, pl.ds(k * 32, 32)]
                        av = bv[e, pl.ds(k * 32, 32)]
                        u0, u1 = plsc.unpack(au, format=plsc.PackFormat.INTERLEAVED)
                        v0, v1 = plsc.unpack(av, format=plsc.PackFormat.INTERLEAVED)
                        acc = acc + u0 * v0
                        acc = acc + u1 * v1
                    vec = jnp.where(lane == j, jnp.sum(acc), vec)
                out_v[pl.ds(pl.multiple_of(cb + gb, 8), LANES)] = vec
                return carry2

            lax.fori_loop(0, CHUNK // LANES, grp_body, 0, unroll=False)

        for s in range(NBUF):
            fire(s, s)

        def ring_body(q, carry):
            c0 = q * NBUF
            for s in range(NBUF):
                drain(s)
                compute(c0 + s, s)

                @pl.when(c0 + s + NBUF < n_chunks)
                def _():
                    fire(c0 + s + NBUF, s)

            return carry

        lax.fori_loop(0, n_chunks // NBUF, ring_body, 0, unroll=False)
        if tail == e_tile:
            pltpu.sync_copy(out_v, out_hbm.at[cid, pl.ds(base, e_tile)])
        else:
            @pl.when(sid < NS - 1)
            def _():
                pltpu.sync_copy(out_v, out_hbm.at[cid, pl.ds(base, e_tile)])

            @pl.when(sid == NS - 1)
            def _():
                pltpu.sync_copy(out_v.at[pl.ds(0, tail)],
                                out_hbm.at[cid, pl.ds(base, tail)])

    return sc_kernel


def _combine_partials(partials, n_edges):
    cols = 128
    rows = n_edges // cols

    def body(p_ref, o_ref):
        o_ref[...] = p_ref[0] + p_ref[1]

    out = pl.pallas_call(
        body,
        out_shape=jax.ShapeDtypeStruct((rows, cols), jnp.float32),
    )(partials.reshape(NC, rows, cols))
    return out.reshape(n_edges, 1)


def kernel(x, edge_index):
    n_nodes, d_model = x.shape
    n_edges = edge_index.shape[1]
    grain = NS * CHUNK * NBUF
    e_pad = ((n_edges + grain - 1) // grain) * grain
    n_feat = d_model // NC

    x_bf = x.astype(jnp.bfloat16)
    ei = edge_index.astype(jnp.int32)

    partials = _make_sc_kernel(n_nodes, n_feat, e_pad, n_edges)(x_bf, ei)
    return _combine_partials(partials, n_edges)
